# baseline (speedup 1.0000x reference)
import functools

import jax
import jax.numpy as jnp
from jax import lax
from jax.experimental import pallas as pl
from jax.experimental.pallas import tpu as pltpu
from jax.experimental.pallas import tpu_sc as plsc

S, D = 2048, 768
E, TOPK, DFF = 8, 2, 3072
N = S * TOPK
BLK = 256
NPAD = N + E * BLK
NB = NPAD // BLK
NW = 32
ROWS_W = NPAD // NW
GCH = 64
TOK_W = S // NW
CCH = 32


def _router_body(x_ref, w_ref, b_ref, o_ref):
    o_ref[...] = jnp.dot(x_ref[...], w_ref[...],
                         preferred_element_type=jnp.float32) + b_ref[...]


def _router(x2, w_pad, b_pad):
    return pl.pallas_call(
        _router_body,
        out_shape=jax.ShapeDtypeStruct((S, 128), jnp.float32),
    )(x2, w_pad, b_pad)


def _ffn_body(be_ref, nb_ref, xs_ref, g_ref, w1_ref, b1_ref, w2_ref, b2_ref,
              o_ref):
    i = pl.program_id(0)

    @pl.when(i < nb_ref[0])
    def _():
        h = jnp.dot(xs_ref[...], w1_ref[0],
                    preferred_element_type=jnp.float32)
        h = jnp.maximum(h + b1_ref[0], 0.0)
        eo = jnp.dot(h, w2_ref[0],
                     preferred_element_type=jnp.float32) + b2_ref[0]
        o_ref[...] = eo * g_ref[:, 0:1]


def _ffn(block_e, nb_act, xs, gates2, W1, b1r, W2, b2r):
    grid_spec = pltpu.PrefetchScalarGridSpec(
        num_scalar_prefetch=2,
        grid=(NB,),
        in_specs=[
            pl.BlockSpec((BLK, D), lambda i, be, nb: (i, 0)),
            pl.BlockSpec((BLK, 128), lambda i, be, nb: (i, 0)),
            pl.BlockSpec((1, D, DFF), lambda i, be, nb: (be[i], 0, 0)),
            pl.BlockSpec((1, 1, DFF), lambda i, be, nb: (be[i], 0, 0)),
            pl.BlockSpec((1, DFF, D), lambda i, be, nb: (be[i], 0, 0)),
            pl.BlockSpec((1, 1, D), lambda i, be, nb: (be[i], 0, 0)),
        ],
        out_specs=pl.BlockSpec((BLK, D), lambda i, be, nb: (i, 0)),
    )
    return pl.pallas_call(
        _ffn_body,
        grid_spec=grid_spec,
        out_shape=jax.ShapeDtypeStruct((NPAD, D), jnp.float32),
    )(block_e, nb_act, xs, gates2, W1, b1r, W2, b2r)


def _sc_gather(x2, tok2):
    mesh = plsc.VectorSubcoreMesh(core_axis_name="c", subcore_axis_name="s")

    @functools.partial(
        pl.kernel, mesh=mesh,
        out_type=jax.ShapeDtypeStruct((NPAD, D), jnp.float32),
        scratch_types=[
            pltpu.VMEM((ROWS_W,), jnp.int32),
            pltpu.VMEM((GCH, D), jnp.float32),
            pltpu.SemaphoreType.DMA,
        ],
    )
    def gk(x_hbm, tok_hbm, xs_hbm, idx_v, buf_v, sem):
        wid = lax.axis_index("s") * 2 + lax.axis_index("c")
        pltpu.sync_copy(tok_hbm.at[pl.ds(wid * ROWS_W, ROWS_W)], idx_v)
        for c in range(ROWS_W // GCH):
            pltpu.async_copy(x_hbm.at[idx_v.at[pl.ds(c * GCH, GCH)]],
                             buf_v, sem).wait()
            pltpu.sync_copy(buf_v,
                            xs_hbm.at[pl.ds(wid * ROWS_W + c * GCH, GCH)])

    return gk(x2, tok2)


def _sc_combine(eo, p0r, p1r):
    mesh = plsc.VectorSubcoreMesh(core_axis_name="c", subcore_axis_name="s")

    @functools.partial(
        pl.kernel, mesh=mesh,
        out_type=jax.ShapeDtypeStruct((S, D), jnp.float32),
        scratch_types=[
            pltpu.VMEM((TOK_W,), jnp.int32),
            pltpu.VMEM((TOK_W,), jnp.int32),
            pltpu.VMEM((CCH, D), jnp.float32),
            pltpu.VMEM((CCH, D), jnp.float32),
            pltpu.SemaphoreType.DMA,
            pltpu.SemaphoreType.DMA,
        ],
    )
    def ck(eo_hbm, p0_hbm, p1_hbm, out_hbm, i0, i1, b0, b1, s0, s1):
        wid = lax.axis_index("s") * 2 + lax.axis_index("c")
        pltpu.sync_copy(p0_hbm.at[pl.ds(wid * TOK_W, TOK_W)], i0)
        pltpu.sync_copy(p1_hbm.at[pl.ds(wid * TOK_W, TOK_W)], i1)
        for c in range(TOK_W // CCH):
            cp0 = pltpu.async_copy(eo_hbm.at[i0.at[pl.ds(c * CCH, CCH)]],
                                   b0, s0)
            cp1 = pltpu.async_copy(eo_hbm.at[i1.at[pl.ds(c * CCH, CCH)]],
                                   b1, s1)
            cp0.wait()
            cp1.wait()

            def row(r, carry):
                for k in range(D // 16):
                    b0[r, pl.ds(k * 16, 16)] = (b0[r, pl.ds(k * 16, 16)]
                                                + b1[r, pl.ds(k * 16, 16)])
                return carry

            lax.fori_loop(0, CCH, row, 0)
            pltpu.sync_copy(b0,
                            out_hbm.at[pl.ds(wid * TOK_W + c * CCH, CCH)])

    return ck(eo, p0r, p1r)


def kernel(x, W_route, b_route, W_noise, b_noise, W1, b1, W2, b2):
    x2 = x.reshape(S, D)
    w_pad = jnp.zeros((D, 128), jnp.float32).at[:, :E].set(W_noise)
    b_pad = jnp.full((1, 128), -1e30, jnp.float32).at[0, :E].set(b_noise)

    nl = _router(x2, w_pad, b_pad)[:, :E]

    tv, ti = lax.top_k(nl, TOPK)
    g = jax.nn.softmax(tv, -1)
    e_f = ti.reshape(-1).astype(jnp.int32)
    g_f = g.reshape(-1)
    t_f = jnp.arange(N, dtype=jnp.int32) // TOPK
    oh = (e_f[:, None] == jnp.arange(E, dtype=jnp.int32)[None, :])
    csum = jnp.cumsum(oh.astype(jnp.int32), 0)
    rank = jnp.sum(csum * oh, 1) - 1
    counts = csum[-1]
    seg = ((counts + BLK - 1) // BLK) * BLK
    cum = jnp.cumsum(seg)
    po = jnp.concatenate([jnp.zeros(1, jnp.int32), cum])
    p_f = po[e_f] + rank
    tok_pad = jnp.zeros(NPAD, jnp.int32).at[p_f].set(t_f)
    gate_pad = jnp.zeros(NPAD, jnp.float32).at[p_f].set(g_f)
    starts = jnp.arange(NB, dtype=jnp.int32) * BLK
    block_e = jnp.minimum(jnp.sum(
        (starts[:, None] >= cum[None, :]).astype(jnp.int32), 1),
        E - 1).astype(jnp.int32)
    nb_act = (cum[-1] // BLK).reshape(1)
    pos = p_f.reshape(S, TOPK)

    xs = _sc_gather(x2, tok_pad)

    gates2 = jnp.broadcast_to(gate_pad[:, None], (NPAD, 128))
    b1r = b1.reshape(E, 1, DFF)
    b2r = b2.reshape(E, 1, D)
    eo = _ffn(block_e, nb_act, xs, gates2, W1, b1r, W2, b2r)

    out2 = _sc_combine(eo, pos[:, 0], pos[:, 1])
    return out2.reshape(1, S, D)


# speedup vs baseline: 1.0038x; 1.0038x over previous
import functools

import jax
import jax.numpy as jnp
from jax import lax
from jax.experimental import pallas as pl
from jax.experimental.pallas import tpu as pltpu
from jax.experimental.pallas import tpu_sc as plsc

S, D = 2048, 768
E, TOPK, DFF = 8, 2, 3072
N = S * TOPK
BLK = 256
NPAD = N + E * BLK
NB = NPAD // BLK
NW = 32
ROWS_W = NPAD // NW
GCH = 64
TOK_W = S // NW
CCH = 32


def _router_body(x_ref, w_ref, b_ref, o_ref):
    o_ref[...] = jnp.dot(x_ref[...], w_ref[...],
                         preferred_element_type=jnp.float32) + b_ref[...]


def _router(x2, w_pad, b_pad):
    return pl.pallas_call(
        _router_body,
        out_shape=jax.ShapeDtypeStruct((S, 128), jnp.float32),
    )(x2, w_pad, b_pad)


def _ffn_body(be_ref, nb_ref, xs_ref, g_ref, w1_ref, b1_ref, w2_ref, b2_ref,
              o_ref):
    i = pl.program_id(0)

    @pl.when(i < nb_ref[0])
    def _():
        h = jnp.dot(xs_ref[...], w1_ref[0],
                    preferred_element_type=jnp.float32)
        h = jnp.maximum(h + b1_ref[0], 0.0)
        eo = jnp.dot(h, w2_ref[0],
                     preferred_element_type=jnp.float32) + b2_ref[0]
        o_ref[...] = eo * g_ref[:, 0:1]


def _ffn(block_e, nb_act, xs, gates2, W1, b1r, W2, b2r):
    grid_spec = pltpu.PrefetchScalarGridSpec(
        num_scalar_prefetch=2,
        grid=(NB,),
        in_specs=[
            pl.BlockSpec((BLK, D), lambda i, be, nb: (i, 0)),
            pl.BlockSpec((BLK, 128), lambda i, be, nb: (i, 0)),
            pl.BlockSpec((1, D, DFF), lambda i, be, nb: (be[i], 0, 0)),
            pl.BlockSpec((1, 1, DFF), lambda i, be, nb: (be[i], 0, 0)),
            pl.BlockSpec((1, DFF, D), lambda i, be, nb: (be[i], 0, 0)),
            pl.BlockSpec((1, 1, D), lambda i, be, nb: (be[i], 0, 0)),
        ],
        out_specs=pl.BlockSpec((BLK, D), lambda i, be, nb: (i, 0)),
    )
    return pl.pallas_call(
        _ffn_body,
        grid_spec=grid_spec,
        out_shape=jax.ShapeDtypeStruct((NPAD, D), jnp.float32),
    )(block_e, nb_act, xs, gates2, W1, b1r, W2, b2r)


def _sc_gather(x2, tok2):
    mesh = plsc.VectorSubcoreMesh(core_axis_name="c", subcore_axis_name="s")

    @functools.partial(
        pl.kernel, mesh=mesh,
        out_type=jax.ShapeDtypeStruct((NPAD, D), jnp.float32),
        scratch_types=[
            pltpu.VMEM((GCH,), jnp.int32),
            pltpu.VMEM((GCH,), jnp.int32),
            pltpu.VMEM((GCH,), jnp.int32),
            pltpu.VMEM((GCH, D), jnp.float32),
            pltpu.VMEM((GCH, D), jnp.float32),
            pltpu.SemaphoreType.DMA,
            pltpu.SemaphoreType.DMA,
        ],
    )
    def gk(x_hbm, tok_hbm, xs_hbm, i0, i1, i2, b0, b1, s0, s1):
        wid = lax.axis_index("s") * 2 + lax.axis_index("c")
        base = wid * ROWS_W
        pltpu.sync_copy(tok_hbm.at[pl.ds(base, GCH)], i0)
        pltpu.sync_copy(tok_hbm.at[pl.ds(base + GCH, GCH)], i1)
        pltpu.sync_copy(tok_hbm.at[pl.ds(base + 2 * GCH, GCH)], i2)
        cp0 = pltpu.async_copy(x_hbm.at[i0], b0, s0)
        cp1 = pltpu.async_copy(x_hbm.at[i1], b1, s1)
        cp0.wait()
        pltpu.sync_copy(b0, xs_hbm.at[pl.ds(base, GCH)])
        cp2 = pltpu.async_copy(x_hbm.at[i2], b0, s0)
        cp1.wait()
        pltpu.sync_copy(b1, xs_hbm.at[pl.ds(base + GCH, GCH)])
        cp2.wait()
        pltpu.sync_copy(b0, xs_hbm.at[pl.ds(base + 2 * GCH, GCH)])

    return gk(x2, tok2)


def _sc_combine(eo, p0r, p1r):
    mesh = plsc.VectorSubcoreMesh(core_axis_name="c", subcore_axis_name="s")

    @functools.partial(
        pl.kernel, mesh=mesh,
        out_type=jax.ShapeDtypeStruct((S, D), jnp.float32),
        scratch_types=[
            pltpu.VMEM((TOK_W,), jnp.int32),
            pltpu.VMEM((TOK_W,), jnp.int32),
            pltpu.VMEM((TOK_W, D), jnp.float32),
            pltpu.VMEM((TOK_W, D), jnp.float32),
            pltpu.SemaphoreType.DMA,
            pltpu.SemaphoreType.DMA,
        ],
    )
    def ck(eo_hbm, p0_hbm, p1_hbm, out_hbm, i0, i1, b0, b1, s0, s1):
        wid = lax.axis_index("s") * 2 + lax.axis_index("c")
        pltpu.sync_copy(p0_hbm.at[pl.ds(wid * TOK_W, TOK_W)], i0)
        pltpu.sync_copy(p1_hbm.at[pl.ds(wid * TOK_W, TOK_W)], i1)
        cp0 = pltpu.async_copy(eo_hbm.at[i0], b0, s0)
        cp1 = pltpu.async_copy(eo_hbm.at[i1], b1, s1)
        cp0.wait()
        cp1.wait()

        def row(r, carry):
            for k in range(D // 16):
                b0[r, pl.ds(k * 16, 16)] = (b0[r, pl.ds(k * 16, 16)]
                                            + b1[r, pl.ds(k * 16, 16)])
            return carry

        lax.fori_loop(0, TOK_W, row, 0)
        pltpu.sync_copy(b0, out_hbm.at[pl.ds(wid * TOK_W, TOK_W)])

    return ck(eo, p0r, p1r)


def kernel(x, W_route, b_route, W_noise, b_noise, W1, b1, W2, b2):
    x2 = x.reshape(S, D)
    w_pad = jnp.zeros((D, 128), jnp.float32).at[:, :E].set(W_noise)
    b_pad = jnp.full((1, 128), -1e30, jnp.float32).at[0, :E].set(b_noise)

    nl = _router(x2, w_pad, b_pad)[:, :E]

    tv, ti = lax.top_k(nl, TOPK)
    g = jax.nn.softmax(tv, -1)
    e_f = ti.reshape(-1).astype(jnp.int32)
    g_f = g.reshape(-1)
    t_f = jnp.arange(N, dtype=jnp.int32) // TOPK
    oh = (e_f[:, None] == jnp.arange(E, dtype=jnp.int32)[None, :])
    csum = jnp.cumsum(oh.astype(jnp.int32), 0)
    rank = jnp.sum(csum * oh, 1) - 1
    counts = csum[-1]
    seg = ((counts + BLK - 1) // BLK) * BLK
    cum = jnp.cumsum(seg)
    po = jnp.concatenate([jnp.zeros(1, jnp.int32), cum])
    p_f = po[e_f] + rank
    tok_pad = jnp.zeros(NPAD, jnp.int32).at[p_f].set(t_f)
    gate_pad = jnp.zeros(NPAD, jnp.float32).at[p_f].set(g_f)
    starts = jnp.arange(NB, dtype=jnp.int32) * BLK
    block_e = jnp.minimum(jnp.sum(
        (starts[:, None] >= cum[None, :]).astype(jnp.int32), 1),
        E - 1).astype(jnp.int32)
    nb_act = (cum[-1] // BLK).reshape(1)
    pos = p_f.reshape(S, TOPK)

    xs = _sc_gather(x2, tok_pad)

    gates2 = jnp.broadcast_to(gate_pad[:, None], (NPAD, 128))
    b1r = b1.reshape(E, 1, DFF)
    b2r = b2.reshape(E, 1, D)
    eo = _ffn(block_e, nb_act, xs, gates2, W1, b1r, W2, b2r)

    out2 = _sc_combine(eo, pos[:, 0], pos[:, 1])
    return out2.reshape(1, S, D)
